# baseline (speedup 1.0000x reference)
import functools

import jax
import jax.numpy as jnp
from jax import lax
from jax.experimental import pallas as pl
from jax.experimental.pallas import tpu as pltpu
from jax.experimental.pallas import tpu_sc as plsc

_N = 10000
_IN = 256
_H = 8
_D = 128
_EFF = _H * _D
_EPS = 1e-5
_BLK = 1000


def _qkv_body(x_ref, nw_ref, nb_ref, nms_ref, wq_ref, bq_ref, wk_ref, bk_ref,
              wv_ref, bv_ref, q_ref, k_ref, v_ref, stat_ref):
    i = pl.program_id(0)

    @pl.when(i == 0)
    def _():
        xf = x_ref[...]
        mean = jnp.mean(xf, axis=0, keepdims=True)
        mu = mean * nms_ref[...]
        var = jnp.mean(xf * xf, axis=0, keepdims=True) - 2.0 * mu * mean + mu * mu
        stat_ref[0:1, :] = mu
        stat_ref[1:2, :] = lax.rsqrt(var + _EPS) * nw_ref[...]

    xb = x_ref[pl.ds(i * _BLK, _BLK), :]
    xn = (xb - stat_ref[0:1, :]) * stat_ref[1:2, :] + nb_ref[...]
    q_ref[...] = jnp.dot(xn, wq_ref[...], preferred_element_type=jnp.float32) + bq_ref[...]
    k_ref[...] = jnp.dot(xn, wk_ref[...], preferred_element_type=jnp.float32) + bk_ref[...]
    v_ref[...] = jnp.dot(xn, wv_ref[...], preferred_element_type=jnp.float32) + bv_ref[...]


def _dense_qkv(x, nq_w, nq_b, nq_ms, Wq, bq, Wk, bk, Wv, bv):
    n = x.shape[0]
    grid = n // _BLK
    full = lambda s: pl.BlockSpec(s, lambda i: (0, 0))
    blk = pl.BlockSpec((_BLK, _EFF), lambda i: (i, 0))
    out = pl.pallas_call(
        _qkv_body,
        grid=(grid,),
        in_specs=[full((n, _IN)), full((1, _IN)), full((1, _IN)), full((1, _IN)),
                  full((_IN, _EFF)), full((1, _EFF)), full((_IN, _EFF)), full((1, _EFF)),
                  full((_IN, _EFF)), full((1, _EFF))],
        out_specs=[blk, blk, blk],
        out_shape=[jax.ShapeDtypeStruct((n, _EFF), jnp.float32)] * 3,
        scratch_shapes=[pltpu.VMEM((2, _IN), jnp.float32)],
    )(x, nq_w.reshape(1, -1), nq_b.reshape(1, -1), nq_ms.reshape(1, -1),
      Wq, bq.reshape(1, -1), Wk, bk.reshape(1, -1), Wv, bv.reshape(1, -1))
    return out


_EPAD = 170496
_WROWS = 320
_SROWS = 40
_NSUB = _WROWS // _SROWS
_NPAD = 32 * _WROWS
_VW = 1152
_DSENT = 1 << 20
_DBLK = 1536
_MCAP = 2048
_INV_SCALE = 1.0 / float(_D) ** 0.5


def _sc_edge_body(q_hbm, k_hbm, v_hbm, src_hbm, dst_hbm, out_hbm,
                  dbuf, sbuf, match_p, qrows, kb, vb, acc,
                  sem2, sem3):
    cid = lax.axis_index("c")
    sid = lax.axis_index("s")
    w = cid * 16 + sid
    iota = lax.iota(jnp.int32, 16)
    lane8 = (iota < 8).astype(jnp.float32)
    zero16 = jnp.zeros((16,), jnp.int32)
    zrow = jnp.zeros((16,), jnp.float32)
    dumpb = lax.broadcast(jnp.int32(_MCAP + 16), (16,))

    def process(cnt, base):
        nch = (cnt + 15) // 16

        def chunk(c, carry):
            bs = c * 16
            pd = match_p[pl.ds(bs, 16)]
            sv = lax.bitwise_and(pd, lax.broadcast(jnp.int32(16383), (16,)))
            dv = lax.shift_right_logical(pd, lax.broadcast(jnp.int32(14), (16,)))
            lv = (lax.broadcast(bs, (16,)) + iota) < lax.broadcast(cnt, (16,))
            sv = jnp.where(lv, sv, zero16)
            cp2 = pltpu.async_copy(k_hbm.at[sv], kb, sem2)
            cp3 = pltpu.async_copy(v_hbm.at[sv], vb, sem3)
            cp2.wait()
            cp3.wait()
            rowv = jnp.where(lv, dv - lax.broadcast(base, (16,)), zero16)

            def pedge(g, c2):
                valid = (bs + g) < cnt

                @pl.when(valid)
                def _():
                    row = jnp.sum(jnp.where(iota == g, rowv, zero16))
                    wv = jnp.zeros((16,), jnp.float32)
                    for h in range(_H):
                        a = (qrows[row, pl.ds(h * _D, 16)] *
                             kb[g, pl.ds(h * _D, 16)])
                        for j in range(1, 8):
                            a = a + (qrows[row, pl.ds(h * _D + j * 16, 16)] *
                                     kb[g, pl.ds(h * _D + j * 16, 16)])
                        sbb = lax.broadcast(jnp.sum(a), (16,))
                        wv = jnp.where(iota == h, sbb, wv)
                    wv = jnp.exp(wv * _INV_SCALE) * lane8
                    for h in range(_H):
                        ws = lax.broadcast(wv[h], (16,))
                        for j in range(8):
                            sl = pl.ds(h * _D + j * 16, 16)
                            acc[row, sl] = acc[row, sl] + vb[g, sl] * ws
                    sl = pl.ds(_EFF, 16)
                    acc[row, sl] = acc[row, sl] + wv

                return c2

            lax.fori_loop(0, 16, pedge, jnp.int32(0))
            return carry

        lax.fori_loop(0, nch, chunk, jnp.int32(0))

    def subround(k, carry0):
        base = w * _WROWS + k * _SROWS
        baseb = lax.broadcast(base, (16,))
        hib = lax.broadcast(base + _SROWS, (16,))
        def zacc(rr, c2):
            for j in range(_VW // 16):
                acc[rr, pl.ds(j * 16, 16)] = zrow
            return c2

        lax.fori_loop(0, _SROWS, zacc, jnp.int32(0))
        pltpu.sync_copy(q_hbm.at[pl.ds(base, _SROWS)], qrows)
        def block(blk, cnt):
            bo = blk * _DBLK
            pltpu.sync_copy(dst_hbm.at[pl.ds(bo, _DBLK)], dbuf)
            pltpu.sync_copy(src_hbm.at[pl.ds(bo, _DBLK)], sbuf)

            def scan(c, cnt):
                dv = dbuf[pl.ds(c * 16, 16)]
                sv = sbuf[pl.ds(c * 16, 16)]
                m = (dv >= baseb) & (dv < hib)
                m01 = m.astype(jnp.int32)
                cs = plsc.cumsum(m01)
                pos = jnp.where(m, lax.broadcast(cnt, (16,)) + cs - m01,
                                dumpb)
                pd = lax.shift_left(dv, lax.broadcast(jnp.int32(14), (16,))) + sv
                plsc.store_scatter(match_p, [pos], pd)
                cnt = cnt + plsc.all_reduce_population_count(m)[0]

                def do_flush(c2):
                    process(c2, base)
                    return jnp.int32(0)

                return lax.cond(cnt >= _MCAP, do_flush, lambda c2: c2, cnt)

            return lax.fori_loop(0, _DBLK // 16, scan, cnt)

        cnt = lax.fori_loop(0, _EPAD // _DBLK, block, jnp.int32(0))
        process(cnt, base)
        pltpu.sync_copy(acc, out_hbm.at[pl.ds(base, _SROWS)])
        return carry0

    lax.fori_loop(0, _NSUB, subround, jnp.int32(0))


def _edge_phase(q, k, v, src, dst):
    npadE = _EPAD - src.shape[0]
    srcp = jnp.concatenate([src, jnp.zeros((npadE,), jnp.int32)])
    dstp = jnp.concatenate([dst, jnp.full((npadE,), _DSENT, jnp.int32)])
    mesh = plsc.VectorSubcoreMesh(core_axis_name="c", subcore_axis_name="s",
                                  num_cores=2, num_subcores=16)
    f = pl.kernel(
        _sc_edge_body,
        out_type=jax.ShapeDtypeStruct((_NPAD, _VW), jnp.float32),
        mesh=mesh,
        compiler_params=pltpu.CompilerParams(needs_layout_passes=False),
        scratch_types=[
            pltpu.VMEM((_DBLK,), jnp.int32),
            pltpu.VMEM((_DBLK,), jnp.int32),
            pltpu.VMEM((_MCAP + 32,), jnp.int32),
            pltpu.VMEM((_SROWS, _EFF), jnp.float32),
            pltpu.VMEM((16, _EFF), jnp.float32),
            pltpu.VMEM((16, _EFF), jnp.float32),
            pltpu.VMEM((_SROWS, _VW), jnp.float32),
            pltpu.SemaphoreType.DMA,
            pltpu.SemaphoreType.DMA,
        ],
    )
    return f(q, k, v, srcp, dstp)


def _combine_body(p_ref, x_ref, h_ref):
    a = p_ref[...]
    acc = jnp.zeros((_BLK, _D), jnp.float32)
    for h in range(_H):
        acc = acc + a[:, h * _D:(h + 1) * _D] / a[:, _EFF + h:_EFF + h + 1]
    h_ref[...] = acc * (1.0 / _H) + x_ref[:, :_D] + x_ref[:, _D:]


def _combine(planes, x):
    n = x.shape[0]
    grid = n // _BLK
    return pl.pallas_call(
        _combine_body,
        grid=(grid,),
        in_specs=[pl.BlockSpec((_BLK, _VW), lambda i: (i, 0)),
                  pl.BlockSpec((_BLK, _IN), lambda i: (i, 0))],
        out_specs=pl.BlockSpec((_BLK, _D), lambda i: (i, 0)),
        out_shape=jax.ShapeDtypeStruct((n, _D), jnp.float32),
    )(planes, x)


def _ffn_body(h_ref, nw_ref, nb_ref, nms_ref, w1_ref, b1_ref, w2_ref, b2_ref, o_ref):
    hf = h_ref[...]
    mean = jnp.mean(hf, axis=0, keepdims=True)
    mu = mean * nms_ref[...]
    var = jnp.mean(hf * hf, axis=0, keepdims=True) - 2.0 * mu * mean + mu * mu
    normed = (hf - mu) * (lax.rsqrt(var + _EPS) * nw_ref[...]) + nb_ref[...]
    z = jnp.dot(normed, w1_ref[...], preferred_element_type=jnp.float32) + b1_ref[...]
    g = z * 0.5 * (1.0 + lax.erf(z * jnp.float32(0.7071067811865476)))
    o_ref[...] = hf + jnp.dot(g, w2_ref[...], preferred_element_type=jnp.float32) + b2_ref[...]


def _norm_ffn(h, no_w, no_b, no_ms, w1, b1, w2, b2):
    n = h.shape[0]
    full2 = lambda s: pl.BlockSpec(s, lambda: (0, 0))
    return pl.pallas_call(
        _ffn_body,
        in_specs=[full2((n, _D)), full2((1, _D)), full2((1, _D)), full2((1, _D)),
                  full2((_D, _D)), full2((1, _D)), full2((_D, _D)), full2((1, _D))],
        out_specs=full2((n, _D)),
        out_shape=jax.ShapeDtypeStruct((n, _D), jnp.float32),
    )(h, no_w.reshape(1, -1), no_b.reshape(1, -1), no_ms.reshape(1, -1),
      w1, b1.reshape(1, -1), w2, b2.reshape(1, -1))


def kernel(x, edge_index, nq_w, nq_b, nq_ms, Wq, bq, Wk, bk, Wv, bv,
           no_w, no_b, no_ms, w1, b1, w2, b2):
    n = x.shape[0]
    q, k, v = _dense_qkv(x, nq_w, nq_b, nq_ms, Wq, bq, Wk, bk, Wv, bv)
    loops = jnp.arange(n, dtype=edge_index.dtype)
    src = jnp.concatenate([edge_index[0], loops])
    dst = jnp.concatenate([edge_index[1], loops])
    planes = _edge_phase(q, k, v, src, dst)
    h = _combine(planes, x)
    return _norm_ffn(h, no_w, no_b, no_ms, w1, b1, w2, b2)


# speedup vs baseline: 1.0430x; 1.0430x over previous
import functools

import jax
import jax.numpy as jnp
from jax import lax
from jax.experimental import pallas as pl
from jax.experimental.pallas import tpu as pltpu
from jax.experimental.pallas import tpu_sc as plsc

_N = 10000
_IN = 256
_H = 8
_D = 128
_EFF = _H * _D
_EPS = 1e-5
_BLK = 1000


def _qkv_body(x_ref, nw_ref, nb_ref, nms_ref, wq_ref, bq_ref, wk_ref, bk_ref,
              wv_ref, bv_ref, q_ref, k_ref, v_ref, stat_ref):
    i = pl.program_id(0)

    @pl.when(i == 0)
    def _():
        xf = x_ref[...]
        mean = jnp.mean(xf, axis=0, keepdims=True)
        mu = mean * nms_ref[...]
        var = jnp.mean(xf * xf, axis=0, keepdims=True) - 2.0 * mu * mean + mu * mu
        stat_ref[0:1, :] = mu
        stat_ref[1:2, :] = lax.rsqrt(var + _EPS) * nw_ref[...]

    xb = x_ref[pl.ds(i * _BLK, _BLK), :]
    xn = (xb - stat_ref[0:1, :]) * stat_ref[1:2, :] + nb_ref[...]
    q_ref[...] = jnp.dot(xn, wq_ref[...], preferred_element_type=jnp.float32) + bq_ref[...]
    k_ref[...] = jnp.dot(xn, wk_ref[...], preferred_element_type=jnp.float32) + bk_ref[...]
    v_ref[...] = jnp.dot(xn, wv_ref[...], preferred_element_type=jnp.float32) + bv_ref[...]


def _dense_qkv(x, nq_w, nq_b, nq_ms, Wq, bq, Wk, bk, Wv, bv):
    n = x.shape[0]
    grid = n // _BLK
    full = lambda s: pl.BlockSpec(s, lambda i: (0, 0))
    blk = pl.BlockSpec((_BLK, _EFF), lambda i: (i, 0))
    out = pl.pallas_call(
        _qkv_body,
        grid=(grid,),
        in_specs=[full((n, _IN)), full((1, _IN)), full((1, _IN)), full((1, _IN)),
                  full((_IN, _EFF)), full((1, _EFF)), full((_IN, _EFF)), full((1, _EFF)),
                  full((_IN, _EFF)), full((1, _EFF))],
        out_specs=[blk, blk, blk],
        out_shape=[jax.ShapeDtypeStruct((n, _EFF), jnp.float32)] * 3,
        scratch_shapes=[pltpu.VMEM((2, _IN), jnp.float32)],
    )(x, nq_w.reshape(1, -1), nq_b.reshape(1, -1), nq_ms.reshape(1, -1),
      Wq, bq.reshape(1, -1), Wk, bk.reshape(1, -1), Wv, bv.reshape(1, -1))
    return out


_EPAD = 170496
_WROWS = 320
_SROWS = 40
_NSUB = _WROWS // _SROWS
_NPAD = 32 * _WROWS
_VW = 1152
_DSENT = 1 << 20
_DBLK = 2304
_MCAP = 2048
_INV_SCALE = 1.0 / float(_D) ** 0.5


def _sc_edge_body(q_hbm, k_hbm, v_hbm, src_hbm, dst_hbm, out_hbm,
                  dbuf, sbuf, match_p, qrows, kb, vb, acc,
                  sem2, sem3):
    cid = lax.axis_index("c")
    sid = lax.axis_index("s")
    w = cid * 16 + sid
    iota = lax.iota(jnp.int32, 16)
    lane8 = (iota < 8).astype(jnp.float32)
    zero16 = jnp.zeros((16,), jnp.int32)
    zrow = jnp.zeros((16,), jnp.float32)
    dumpb = lax.broadcast(jnp.int32(_MCAP + 16), (16,))

    def process(cnt, base):
        nch = (cnt + 15) // 16

        def chunk(c, carry):
            bs = c * 16
            pd = match_p[pl.ds(bs, 16)]
            sv = lax.bitwise_and(pd, lax.broadcast(jnp.int32(16383), (16,)))
            dv = lax.shift_right_logical(pd, lax.broadcast(jnp.int32(14), (16,)))
            lv = (lax.broadcast(bs, (16,)) + iota) < lax.broadcast(cnt, (16,))
            sv = jnp.where(lv, sv, zero16)
            cp2 = pltpu.async_copy(k_hbm.at[sv], kb, sem2)
            cp3 = pltpu.async_copy(v_hbm.at[sv], vb, sem3)
            cp2.wait()
            cp3.wait()
            rowv = jnp.where(lv, dv - lax.broadcast(base, (16,)), zero16)

            def pedge(g, c2):
                valid = (bs + g) < cnt

                @pl.when(valid)
                def _():
                    row = jnp.sum(jnp.where(iota == g, rowv, zero16))
                    wv = jnp.zeros((16,), jnp.float32)
                    for h in range(_H):
                        a = (qrows[row, pl.ds(h * _D, 16)] *
                             kb[g, pl.ds(h * _D, 16)])
                        for j in range(1, 8):
                            a = a + (qrows[row, pl.ds(h * _D + j * 16, 16)] *
                                     kb[g, pl.ds(h * _D + j * 16, 16)])
                        sbb = lax.broadcast(jnp.sum(a), (16,))
                        wv = jnp.where(iota == h, sbb, wv)
                    wv = jnp.exp(wv * _INV_SCALE) * lane8
                    for h in range(_H):
                        ws = lax.broadcast(wv[h], (16,))
                        for j in range(8):
                            sl = pl.ds(h * _D + j * 16, 16)
                            acc[row, sl] = acc[row, sl] + vb[g, sl] * ws
                    sl = pl.ds(_EFF, 16)
                    acc[row, sl] = acc[row, sl] + wv

                return c2

            lax.fori_loop(0, 16, pedge, jnp.int32(0))
            return carry

        lax.fori_loop(0, nch, chunk, jnp.int32(0))

    def subround(k, carry0):
        base = w * _WROWS + k * _SROWS
        baseb = lax.broadcast(base, (16,))
        hib = lax.broadcast(base + _SROWS, (16,))
        def zacc(rr, c2):
            for j in range(_VW // 16):
                acc[rr, pl.ds(j * 16, 16)] = zrow
            return c2

        lax.fori_loop(0, _SROWS, zacc, jnp.int32(0))
        pltpu.sync_copy(q_hbm.at[pl.ds(base, _SROWS)], qrows)
        def block(blk, cnt):
            bo = blk * _DBLK
            pltpu.sync_copy(dst_hbm.at[pl.ds(bo, _DBLK)], dbuf)
            pltpu.sync_copy(src_hbm.at[pl.ds(bo, _DBLK)], sbuf)

            def scan(c, cnt):
                dv = dbuf[pl.ds(c * 16, 16)]
                sv = sbuf[pl.ds(c * 16, 16)]
                m = (dv >= baseb) & (dv < hib)
                m01 = m.astype(jnp.int32)
                cs = plsc.cumsum(m01)
                pos = jnp.where(m, lax.broadcast(cnt, (16,)) + cs - m01,
                                dumpb)
                pd = lax.shift_left(dv, lax.broadcast(jnp.int32(14), (16,))) + sv
                plsc.store_scatter(match_p, [pos], pd)
                cnt = cnt + plsc.all_reduce_population_count(m)[0]

                def do_flush(c2):
                    process(c2, base)
                    return jnp.int32(0)

                return lax.cond(cnt >= _MCAP, do_flush, lambda c2: c2, cnt)

            return lax.fori_loop(0, _DBLK // 16, scan, cnt)

        cnt = lax.fori_loop(0, _EPAD // _DBLK, block, jnp.int32(0))
        process(cnt, base)
        pltpu.sync_copy(acc, out_hbm.at[pl.ds(base, _SROWS)])
        return carry0

    lax.fori_loop(0, _NSUB, subround, jnp.int32(0))


def _edge_phase(q, k, v, src, dst):
    npadE = _EPAD - src.shape[0]
    srcp = jnp.concatenate([src, jnp.zeros((npadE,), jnp.int32)])
    dstp = jnp.concatenate([dst, jnp.full((npadE,), _DSENT, jnp.int32)])
    mesh = plsc.VectorSubcoreMesh(core_axis_name="c", subcore_axis_name="s",
                                  num_cores=2, num_subcores=16)
    f = pl.kernel(
        _sc_edge_body,
        out_type=jax.ShapeDtypeStruct((_NPAD, _VW), jnp.float32),
        mesh=mesh,
        compiler_params=pltpu.CompilerParams(needs_layout_passes=False),
        scratch_types=[
            pltpu.VMEM((_DBLK,), jnp.int32),
            pltpu.VMEM((_DBLK,), jnp.int32),
            pltpu.VMEM((_MCAP + 32,), jnp.int32),
            pltpu.VMEM((_SROWS, _EFF), jnp.float32),
            pltpu.VMEM((16, _EFF), jnp.float32),
            pltpu.VMEM((16, _EFF), jnp.float32),
            pltpu.VMEM((_SROWS, _VW), jnp.float32),
            pltpu.SemaphoreType.DMA,
            pltpu.SemaphoreType.DMA,
        ],
    )
    return f(q, k, v, srcp, dstp)


def _combine_body(p_ref, x_ref, h_ref):
    a = p_ref[...]
    acc = jnp.zeros((_BLK, _D), jnp.float32)
    for h in range(_H):
        acc = acc + a[:, h * _D:(h + 1) * _D] / a[:, _EFF + h:_EFF + h + 1]
    h_ref[...] = acc * (1.0 / _H) + x_ref[:, :_D] + x_ref[:, _D:]


def _combine(planes, x):
    n = x.shape[0]
    grid = n // _BLK
    return pl.pallas_call(
        _combine_body,
        grid=(grid,),
        in_specs=[pl.BlockSpec((_BLK, _VW), lambda i: (i, 0)),
                  pl.BlockSpec((_BLK, _IN), lambda i: (i, 0))],
        out_specs=pl.BlockSpec((_BLK, _D), lambda i: (i, 0)),
        out_shape=jax.ShapeDtypeStruct((n, _D), jnp.float32),
    )(planes, x)


def _ffn_body(h_ref, nw_ref, nb_ref, nms_ref, w1_ref, b1_ref, w2_ref, b2_ref, o_ref):
    hf = h_ref[...]
    mean = jnp.mean(hf, axis=0, keepdims=True)
    mu = mean * nms_ref[...]
    var = jnp.mean(hf * hf, axis=0, keepdims=True) - 2.0 * mu * mean + mu * mu
    normed = (hf - mu) * (lax.rsqrt(var + _EPS) * nw_ref[...]) + nb_ref[...]
    z = jnp.dot(normed, w1_ref[...], preferred_element_type=jnp.float32) + b1_ref[...]
    g = z * 0.5 * (1.0 + lax.erf(z * jnp.float32(0.7071067811865476)))
    o_ref[...] = hf + jnp.dot(g, w2_ref[...], preferred_element_type=jnp.float32) + b2_ref[...]


def _norm_ffn(h, no_w, no_b, no_ms, w1, b1, w2, b2):
    n = h.shape[0]
    full2 = lambda s: pl.BlockSpec(s, lambda: (0, 0))
    return pl.pallas_call(
        _ffn_body,
        in_specs=[full2((n, _D)), full2((1, _D)), full2((1, _D)), full2((1, _D)),
                  full2((_D, _D)), full2((1, _D)), full2((_D, _D)), full2((1, _D))],
        out_specs=full2((n, _D)),
        out_shape=jax.ShapeDtypeStruct((n, _D), jnp.float32),
    )(h, no_w.reshape(1, -1), no_b.reshape(1, -1), no_ms.reshape(1, -1),
      w1, b1.reshape(1, -1), w2, b2.reshape(1, -1))


def kernel(x, edge_index, nq_w, nq_b, nq_ms, Wq, bq, Wk, bk, Wv, bv,
           no_w, no_b, no_ms, w1, b1, w2, b2):
    n = x.shape[0]
    q, k, v = _dense_qkv(x, nq_w, nq_b, nq_ms, Wq, bq, Wk, bk, Wv, bv)
    loops = jnp.arange(n, dtype=edge_index.dtype)
    src = jnp.concatenate([edge_index[0], loops])
    dst = jnp.concatenate([edge_index[1], loops])
    planes = _edge_phase(q, k, v, src, dst)
    h = _combine(planes, x)
    return _norm_ffn(h, no_w, no_b, no_ms, w1, b1, w2, b2)


# speedup vs baseline: 1.0603x; 1.0166x over previous
import functools

import jax
import jax.numpy as jnp
from jax import lax
from jax.experimental import pallas as pl
from jax.experimental.pallas import tpu as pltpu
from jax.experimental.pallas import tpu_sc as plsc

_N = 10000
_IN = 256
_H = 8
_D = 128
_EFF = _H * _D
_EPS = 1e-5
_BLK = 1000


def _qkv_body(x_ref, nw_ref, nb_ref, nms_ref, wq_ref, bq_ref, wk_ref, bk_ref,
              wv_ref, bv_ref, q_ref, k_ref, v_ref, stat_ref):
    i = pl.program_id(0)

    @pl.when(i == 0)
    def _():
        xf = x_ref[...]
        mean = jnp.mean(xf, axis=0, keepdims=True)
        mu = mean * nms_ref[...]
        var = jnp.mean(xf * xf, axis=0, keepdims=True) - 2.0 * mu * mean + mu * mu
        stat_ref[0:1, :] = mu
        stat_ref[1:2, :] = lax.rsqrt(var + _EPS) * nw_ref[...]

    xb = x_ref[pl.ds(i * _BLK, _BLK), :]
    xn = (xb - stat_ref[0:1, :]) * stat_ref[1:2, :] + nb_ref[...]
    q_ref[...] = jnp.dot(xn, wq_ref[...], preferred_element_type=jnp.float32) + bq_ref[...]
    k_ref[...] = jnp.dot(xn, wk_ref[...], preferred_element_type=jnp.float32) + bk_ref[...]
    v_ref[:, :_EFF] = jnp.dot(xn, wv_ref[...], preferred_element_type=jnp.float32) + bv_ref[...]
    v_ref[:, _EFF:] = jnp.zeros((_BLK, 128), jnp.float32)


def _dense_qkv(x, nq_w, nq_b, nq_ms, Wq, bq, Wk, bk, Wv, bv):
    n = x.shape[0]
    grid = n // _BLK
    full = lambda s: pl.BlockSpec(s, lambda i: (0, 0))
    blk = pl.BlockSpec((_BLK, _EFF), lambda i: (i, 0))
    out = pl.pallas_call(
        _qkv_body,
        grid=(grid,),
        in_specs=[full((n, _IN)), full((1, _IN)), full((1, _IN)), full((1, _IN)),
                  full((_IN, _EFF)), full((1, _EFF)), full((_IN, _EFF)), full((1, _EFF)),
                  full((_IN, _EFF)), full((1, _EFF))],
        out_specs=[blk, blk, pl.BlockSpec((_BLK, 1152), lambda i: (i, 0))],
        out_shape=[jax.ShapeDtypeStruct((n, _EFF), jnp.float32),
                   jax.ShapeDtypeStruct((n, _EFF), jnp.float32),
                   jax.ShapeDtypeStruct((n, 1152), jnp.float32)],
        scratch_shapes=[pltpu.VMEM((2, _IN), jnp.float32)],
    )(x, nq_w.reshape(1, -1), nq_b.reshape(1, -1), nq_ms.reshape(1, -1),
      Wq, bq.reshape(1, -1), Wk, bk.reshape(1, -1), Wv, bv.reshape(1, -1))
    return out


_EPAD = 170496
_WROWS = 320
_SROWS = 40
_NSUB = _WROWS // _SROWS
_NPAD = 32 * _WROWS
_VW = 1152
_DSENT = 1 << 20
_DBLK = 4608
_MCAP = 2048
_INV_SCALE = 1.0 / float(_D) ** 0.5


def _sc_edge_body(q_hbm, k_hbm, v_hbm, src_hbm, dst_hbm, out_hbm,
                  dbuf, sbuf, match_p, qb, kb, vb, acc,
                  sem1, sem2, sem3):
    cid = lax.axis_index("c")
    sid = lax.axis_index("s")
    w = cid * 16 + sid
    iota = lax.iota(jnp.int32, 16)
    lane8 = (iota < 8).astype(jnp.float32)
    zero16 = jnp.zeros((16,), jnp.int32)
    zrow = jnp.zeros((16,), jnp.float32)
    dumpb = lax.broadcast(jnp.int32(_MCAP + 16), (16,))

    def process(cnt, base):
        nch = (cnt + 15) // 16

        def chunk(c, carry):
            bs = c * 16
            pd = match_p[pl.ds(bs, 16)]
            sv = lax.bitwise_and(pd, lax.broadcast(jnp.int32(16383), (16,)))
            dv = lax.shift_right_logical(pd, lax.broadcast(jnp.int32(14), (16,)))
            lv = (lax.broadcast(bs, (16,)) + iota) < lax.broadcast(cnt, (16,))
            sv = jnp.where(lv, sv, zero16)
            dvg = jnp.where(lv, dv, zero16)
            cp1 = pltpu.async_copy(q_hbm.at[dvg], qb, sem1)
            cp2 = pltpu.async_copy(k_hbm.at[sv], kb, sem2)
            cp3 = pltpu.async_copy(v_hbm.at[sv], vb, sem3)
            cp1.wait()
            cp2.wait()
            cp3.wait()
            rowv = jnp.where(lv, dv - lax.broadcast(base, (16,)), zero16)

            def pedge(g, c2):
                valid = (bs + g) < cnt

                @pl.when(valid)
                def _():
                    row = jnp.sum(jnp.where(iota == g, rowv, zero16))
                    wv = jnp.zeros((16,), jnp.float32)
                    for h in range(_H):
                        a = (qb[g, pl.ds(h * _D, 16)] *
                             kb[g, pl.ds(h * _D, 16)])
                        for j in range(1, 8):
                            a = a + (qb[g, pl.ds(h * _D + j * 16, 16)] *
                                     kb[g, pl.ds(h * _D + j * 16, 16)])
                        sbb = lax.broadcast(jnp.sum(a), (16,))
                        wv = jnp.where(iota == h, sbb, wv)
                    wv = jnp.exp(wv * _INV_SCALE) * lane8
                    for h in range(_H):
                        ws = lax.broadcast(wv[h], (16,))
                        for j in range(8):
                            sl = pl.ds(h * _D + j * 16, 16)
                            acc[row, sl] = acc[row, sl] + vb[g, sl] * ws
                    sl = pl.ds(_EFF, 16)
                    acc[row, sl] = acc[row, sl] + wv

                return c2

            lax.fori_loop(0, 16, pedge, jnp.int32(0))
            return carry

        lax.fori_loop(0, nch, chunk, jnp.int32(0))

    def subround(k, carry0):
        base = w * _WROWS + k * _SROWS
        baseb = lax.broadcast(base, (16,))
        hib = lax.broadcast(base + _SROWS, (16,))
        def zacc(rr, c2):
            for j in range(_VW // 16):
                acc[rr, pl.ds(j * 16, 16)] = zrow
            return c2

        lax.fori_loop(0, _SROWS, zacc, jnp.int32(0))
        def block(blk, cnt):
            bo = blk * _DBLK
            pltpu.sync_copy(dst_hbm.at[pl.ds(bo, _DBLK)], dbuf)
            pltpu.sync_copy(src_hbm.at[pl.ds(bo, _DBLK)], sbuf)

            def scan(c, cnt):
                dv = dbuf[pl.ds(c * 16, 16)]
                sv = sbuf[pl.ds(c * 16, 16)]
                m = (dv >= baseb) & (dv < hib)
                m01 = m.astype(jnp.int32)
                cs = plsc.cumsum(m01)
                pos = jnp.where(m, lax.broadcast(cnt, (16,)) + cs - m01,
                                dumpb)
                pd = lax.shift_left(dv, lax.broadcast(jnp.int32(14), (16,))) + sv
                plsc.store_scatter(match_p, [pos], pd)
                cnt = cnt + plsc.all_reduce_population_count(m)[0]

                def do_flush(c2):
                    process(c2, base)
                    return jnp.int32(0)

                return lax.cond(cnt >= _MCAP, do_flush, lambda c2: c2, cnt)

            return lax.fori_loop(0, _DBLK // 16, scan, cnt)

        cnt = lax.fori_loop(0, _EPAD // _DBLK, block, jnp.int32(0))
        process(cnt, base)
        pltpu.sync_copy(acc, out_hbm.at[pl.ds(base, _SROWS)])
        return carry0

    lax.fori_loop(0, _NSUB, subround, jnp.int32(0))


def _edge_phase(q, k, v, src, dst):
    npadE = _EPAD - src.shape[0]
    srcp = jnp.concatenate([src, jnp.zeros((npadE,), jnp.int32)])
    dstp = jnp.concatenate([dst, jnp.full((npadE,), _DSENT, jnp.int32)])
    mesh = plsc.VectorSubcoreMesh(core_axis_name="c", subcore_axis_name="s",
                                  num_cores=2, num_subcores=16)
    f = pl.kernel(
        _sc_edge_body,
        out_type=jax.ShapeDtypeStruct((_NPAD, _VW), jnp.float32),
        mesh=mesh,
        compiler_params=pltpu.CompilerParams(needs_layout_passes=False),
        scratch_types=[
            pltpu.VMEM((_DBLK,), jnp.int32),
            pltpu.VMEM((_DBLK,), jnp.int32),
            pltpu.VMEM((_MCAP + 32,), jnp.int32),
            pltpu.VMEM((16, _EFF), jnp.float32),
            pltpu.VMEM((16, _EFF), jnp.float32),
            pltpu.VMEM((16, _VW), jnp.float32),
            pltpu.VMEM((_SROWS, _VW), jnp.float32),
            pltpu.SemaphoreType.DMA,
            pltpu.SemaphoreType.DMA,
            pltpu.SemaphoreType.DMA,
        ],
    )
    return f(q, k, v, srcp, dstp)


def _combine_body(p_ref, x_ref, h_ref):
    a = p_ref[...]
    acc = jnp.zeros((_BLK, _D), jnp.float32)
    for h in range(_H):
        acc = acc + a[:, h * _D:(h + 1) * _D] / a[:, _EFF + h:_EFF + h + 1]
    h_ref[...] = acc * (1.0 / _H) + x_ref[:, :_D] + x_ref[:, _D:]


def _combine(planes, x):
    n = x.shape[0]
    grid = n // _BLK
    return pl.pallas_call(
        _combine_body,
        grid=(grid,),
        in_specs=[pl.BlockSpec((_BLK, _VW), lambda i: (i, 0)),
                  pl.BlockSpec((_BLK, _IN), lambda i: (i, 0))],
        out_specs=pl.BlockSpec((_BLK, _D), lambda i: (i, 0)),
        out_shape=jax.ShapeDtypeStruct((n, _D), jnp.float32),
    )(planes, x)


def _ffn_body(h_ref, nw_ref, nb_ref, nms_ref, w1_ref, b1_ref, w2_ref, b2_ref, o_ref):
    hf = h_ref[...]
    mean = jnp.mean(hf, axis=0, keepdims=True)
    mu = mean * nms_ref[...]
    var = jnp.mean(hf * hf, axis=0, keepdims=True) - 2.0 * mu * mean + mu * mu
    normed = (hf - mu) * (lax.rsqrt(var + _EPS) * nw_ref[...]) + nb_ref[...]
    z = jnp.dot(normed, w1_ref[...], preferred_element_type=jnp.float32) + b1_ref[...]
    g = z * 0.5 * (1.0 + lax.erf(z * jnp.float32(0.7071067811865476)))
    o_ref[...] = hf + jnp.dot(g, w2_ref[...], preferred_element_type=jnp.float32) + b2_ref[...]


def _norm_ffn(h, no_w, no_b, no_ms, w1, b1, w2, b2):
    n = h.shape[0]
    full2 = lambda s: pl.BlockSpec(s, lambda: (0, 0))
    return pl.pallas_call(
        _ffn_body,
        in_specs=[full2((n, _D)), full2((1, _D)), full2((1, _D)), full2((1, _D)),
                  full2((_D, _D)), full2((1, _D)), full2((_D, _D)), full2((1, _D))],
        out_specs=full2((n, _D)),
        out_shape=jax.ShapeDtypeStruct((n, _D), jnp.float32),
    )(h, no_w.reshape(1, -1), no_b.reshape(1, -1), no_ms.reshape(1, -1),
      w1, b1.reshape(1, -1), w2, b2.reshape(1, -1))


def kernel(x, edge_index, nq_w, nq_b, nq_ms, Wq, bq, Wk, bk, Wv, bv,
           no_w, no_b, no_ms, w1, b1, w2, b2):
    n = x.shape[0]
    q, k, v = _dense_qkv(x, nq_w, nq_b, nq_ms, Wq, bq, Wk, bk, Wv, bv)
    loops = jnp.arange(n, dtype=edge_index.dtype)
    src = jnp.concatenate([edge_index[0], loops])
    dst = jnp.concatenate([edge_index[1], loops])
    planes = _edge_phase(q, k, v, src, dst)
    h = _combine(planes, x)
    return _norm_ffn(h, no_w, no_b, no_ms, w1, b1, w2, b2)
